# baseline (speedup 1.0000x reference)
import jax
import jax.numpy as jnp
from jax.experimental import pallas as pl
from jax.experimental.pallas import tpu as pltpu

_TOKENS = 16384
_HIDDEN = 2048
_E = 16
_CHUNK = 2048
_NBUF = 2
_NCH = _TOKENS // _CHUNK


def _t2(a):
    pad = jnp.zeros((6, a.shape[1]), a.dtype)
    return jnp.transpose(jnp.concatenate([a, pad], axis=0))[:, :2]


def _router_body(x_hbm, w_ref, brow_ref,
                 logits_hbm, wts_hbm, idx_hbm, mask_ref,
                 xbuf, lbuf, wbuf, ibuf, xsems, osems):
    def xcopy(c, slot):
        return pltpu.make_async_copy(
            x_hbm.at[pl.ds(c * _CHUNK, _CHUNK), :],
            xbuf.at[slot], xsems.at[slot])

    def ocopies(c, oslot):
        tok = pl.ds(c * _CHUNK, _CHUNK)
        return (
            pltpu.make_async_copy(lbuf.at[oslot], logits_hbm.at[tok, :],
                                  osems.at[oslot, 0]),
            pltpu.make_async_copy(wbuf.at[oslot], wts_hbm.at[tok, :],
                                  osems.at[oslot, 1]),
            pltpu.make_async_copy(ibuf.at[oslot], idx_hbm.at[tok, :],
                                  osems.at[oslot, 2]),
        )

    for i in range(min(_NBUF, _NCH)):
        xcopy(i, i).start()
    w = w_ref[...]
    brow = brow_ref[...]

    for c in range(_NCH):
        slot = c % _NBUF
        oslot = c % 2
        xcopy(c, slot).wait()
        x = xbuf[slot]
        if c >= 2:
            for cp in ocopies(c - 2, oslot):
                cp.wait()
        lbuf[oslot] = jax.lax.dot_general(
            x, w, (((1,), (1,)), ((), ())),
            preferred_element_type=jnp.float32) + brow
        nxt = c + _NBUF
        if nxt < _NCH:
            xcopy(nxt, slot).start()

        lt = jnp.transpose(lbuf[oslot])
        m = jnp.max(lt, axis=0, keepdims=True)
        ex = jnp.exp(lt - m)
        p = ex / jnp.sum(ex, axis=0, keepdims=True)

        iota = jax.lax.broadcasted_iota(jnp.int32, p.shape, 0)
        p1 = jnp.max(p, axis=0, keepdims=True)
        i1 = jnp.min(jnp.where(p == p1, iota, _E), axis=0, keepdims=True)
        oh1 = (iota == i1)
        pm = jnp.where(oh1, -1.0, p)
        p2 = jnp.max(pm, axis=0, keepdims=True)
        i2 = jnp.min(jnp.where(pm == p2, iota, _E), axis=0, keepdims=True)
        oh2 = (iota == i2)

        tok = pl.ds(c * _CHUNK, _CHUNK)
        mask_ref[:, 0, tok] = oh1.astype(jnp.int32)
        mask_ref[:, 1, tok] = oh2.astype(jnp.int32)

        s = p1 + p2
        wbuf[oslot] = _t2(jnp.concatenate([p1 / s, p2 / s], axis=0))
        idxf = jnp.concatenate([i1, i2], axis=0).astype(jnp.float32)
        ibuf[oslot] = _t2(idxf).astype(jnp.int32)
        for cp in ocopies(c, oslot):
            cp.start()

    for c in (_NCH - 2, _NCH - 1):
        for cp in ocopies(c, c % 2):
            cp.wait()


def kernel(x, gate_w, gate_b):
    brow = gate_b.reshape(1, _E)
    hbm = pl.BlockSpec(memory_space=pltpu.MemorySpace.HBM)
    vmem = pl.BlockSpec(memory_space=pltpu.MemorySpace.VMEM)
    logits, wts, idx, mask = pl.pallas_call(
        _router_body,
        in_specs=[hbm, vmem, vmem],
        out_specs=[hbm, hbm, hbm, vmem],
        out_shape=[
            jax.ShapeDtypeStruct((_TOKENS, _E), jnp.float32),
            jax.ShapeDtypeStruct((_TOKENS, 2), jnp.float32),
            jax.ShapeDtypeStruct((_TOKENS, 2), jnp.int32),
            jax.ShapeDtypeStruct((_E, 2, _TOKENS), jnp.int32),
        ],
        scratch_shapes=[
            pltpu.VMEM((_NBUF, _CHUNK, _HIDDEN), jnp.float32),
            pltpu.VMEM((2, _CHUNK, _E), jnp.float32),
            pltpu.VMEM((2, _CHUNK, 2), jnp.float32),
            pltpu.VMEM((2, _CHUNK, 2), jnp.int32),
            pltpu.SemaphoreType.DMA((_NBUF,)),
            pltpu.SemaphoreType.DMA((2, 3)),
        ],
    )(x, gate_w, brow)
    return (logits, wts, idx, mask)
